# baseline (speedup 1.0000x reference)
import functools

import jax
import jax.numpy as jnp
from jax.experimental import pallas as pl

B = 64
N = 2048
NPATCH = 4
NSAMPLE = 256
NOUT = 16
GROUP = 7
BISECT_ITERS = 50
F32 = jnp.float32


def _bisect_kth(dvals, k, iters=BISECT_ITERS):
    lo = jnp.zeros((dvals.shape[0], 1), dtype=F32)
    hi = jnp.max(dvals, axis=1, keepdims=True) + 1.0

    def body(_, carry):
        lo, hi = carry
        mid = 0.5 * (lo + hi)
        cnt = jnp.sum((dvals <= mid).astype(F32), axis=1, keepdims=True)
        pred = cnt >= k
        return (jnp.where(pred, lo, mid), jnp.where(pred, mid, hi))

    lo, hi = jax.lax.fori_loop(0, iters, body, (lo, hi))
    return hi


def _samplenet_kernel(xyz_ref,
                      w0t, b0c, w1t, b1c, w2t, b2c, w3t, b3c, w4t, b4c,
                      v0t, c0c, v1t, c1c, v2t, c2c, wxt, wyt, wzt,
                      bxc, byc, bzc, tt_ref,
                      proj_ref, simp_ref, seeds_ref):
    (w0t, b0c, w1t, b1c, w2t, b2c, w3t, b3c, w4t, b4c,
     v0t, c0c, v1t, c1c, v2t, c2c, wxt, wyt, wzt, bxc, byc, bzc) = (
        r[...] for r in (w0t, b0c, w1t, b1c, w2t, b2c, w3t, b3c, w4t, b4c,
                         v0t, c0c, v1t, c1c, v2t, c2c, wxt, wyt, wzt,
                         bxc, byc, bzc))
    x = xyz_ref[0, 0:1, :]
    y = xyz_ref[0, 1:2, :]
    z = xyz_ref[0, 2:3, :]
    iota = jax.lax.broadcasted_iota(jnp.int32, (1, N), 1)

    oh = (iota == 0).astype(F32)
    sel = [oh]
    lx = jnp.sum(x * oh, axis=1, keepdims=True)
    ly = jnp.sum(y * oh, axis=1, keepdims=True)
    lz = jnp.sum(z * oh, axis=1, keepdims=True)
    dists = jnp.full((1, N), 1e10, dtype=F32)
    for _ in range(NPATCH - 1):
        d = (x - lx) ** 2 + (y - ly) ** 2 + (z - lz) ** 2
        dists = jnp.minimum(dists, d)
        m = jnp.max(dists, axis=1, keepdims=True)
        idx = jnp.min(jnp.where(dists >= m, iota, N), axis=1, keepdims=True)
        oh = (iota == idx).astype(F32)
        sel.append(oh)
        lx = jnp.sum(x * oh, axis=1, keepdims=True)
        ly = jnp.sum(y * oh, axis=1, keepdims=True)
        lz = jnp.sum(z * oh, axis=1, keepdims=True)
    selm = jnp.concatenate(sel, axis=0)
    sx = jnp.sum(selm * x, axis=1, keepdims=True)
    sy = jnp.sum(selm * y, axis=1, keepdims=True)
    sz = jnp.sum(selm * z, axis=1, keepdims=True)
    seeds_ref[0] = jnp.concatenate([sx, sy, sz], axis=1)

    d2 = (sx - x) ** 2 + (sy - y) ** 2 + (sz - z) ** 2
    thr = _bisect_kth(d2, NSAMPLE)
    mask = (d2 <= thr).astype(F32)

    inv = 1.0 / NSAMPLE
    mx = jnp.sum(mask * x, axis=1, keepdims=True) * inv
    my = jnp.sum(mask * y, axis=1, keepdims=True) * inv
    mz = jnp.sum(mask * z, axis=1, keepdims=True) * inv
    ex = x - mx
    ey = y - my
    ez = z - mz
    r2 = ex * ex + ey * ey + ez * ez
    norm = jnp.sqrt(jnp.max(jnp.where(mask > 0, r2, 0.0), axis=1,
                            keepdims=True) + 1e-12)
    xn = ex / norm
    yn = ey / norm
    zn = ez / norm

    tt = tt_ref[0, 0]

    pooled = []
    for p in range(NPATCH):
        xr = xn[p:p + 1, :]
        yr = yn[p:p + 1, :]
        zr = zn[p:p + 1, :]
        h = (w0t[:, 0:1] * xr + w0t[:, 1:2] * yr + w0t[:, 2:3] * zr + b0c)
        h = jnp.maximum(h, 0.0)
        h = jnp.maximum(jnp.dot(w1t, h, preferred_element_type=F32) + b1c, 0.0)
        h = jnp.maximum(jnp.dot(w2t, h, preferred_element_type=F32) + b2c, 0.0)
        h = jnp.maximum(jnp.dot(w3t, h, preferred_element_type=F32) + b3c, 0.0)
        h = jnp.maximum(jnp.dot(w4t, h, preferred_element_type=F32) + b4c, 0.0)
        pooled.append(jnp.max(jnp.where(mask[p:p + 1, :] > 0, h, 0.0),
                              axis=1, keepdims=True))
    f = jnp.concatenate(pooled, axis=1)

    g = jnp.maximum(jnp.dot(v0t, f, preferred_element_type=F32) + c0c, 0.0)
    g = jnp.maximum(jnp.dot(v1t, g, preferred_element_type=F32) + c1c, 0.0)
    g = jnp.maximum(jnp.dot(v2t, g, preferred_element_type=F32) + c2c, 0.0)
    qx = jnp.dot(wxt, g, preferred_element_type=F32) + bxc
    qy = jnp.dot(wyt, g, preferred_element_type=F32) + byc
    qz = jnp.dot(wzt, g, preferred_element_type=F32) + bzc

    for p in range(NPATCH):
        qxp = qx[:, p:p + 1]
        qyp = qy[:, p:p + 1]
        qzp = qz[:, p:p + 1]
        mp = mask[p:p + 1, :]
        xr = xn[p:p + 1, :]
        yr = yn[p:p + 1, :]
        zr = zn[p:p + 1, :]
        d2p = (qxp - xr) ** 2 + (qyp - yr) ** 2 + (qzp - zr) ** 2
        dm = jnp.where(mp > 0, d2p, 1e30)
        lo = jnp.zeros((NOUT, 1), dtype=F32)
        hi = jnp.max(jnp.where(mp > 0, d2p, 0.0), axis=1, keepdims=True) + 1.0

        def body(_, carry, dm=dm):
            lo, hi = carry
            mid = 0.5 * (lo + hi)
            cnt = jnp.sum((dm <= mid).astype(F32), axis=1, keepdims=True)
            pred = cnt >= GROUP
            return (jnp.where(pred, lo, mid), jnp.where(pred, mid, hi))

        lo, hi = jax.lax.fori_loop(0, BISECT_ITERS, body, (lo, hi))
        gm = dm <= hi
        dmin = jnp.min(dm, axis=1, keepdims=True)
        e = jnp.where(gm, jnp.exp((dmin - d2p) / tt), 0.0)
        s = jnp.sum(e, axis=1, keepdims=True)
        wgt = e / s
        prx = jnp.sum(wgt * xr, axis=1, keepdims=True)
        pry = jnp.sum(wgt * yr, axis=1, keepdims=True)
        prz = jnp.sum(wgt * zr, axis=1, keepdims=True)
        np_ = norm[p:p + 1, :]
        mxp = mx[p:p + 1, :]
        myp = my[p:p + 1, :]
        mzp = mz[p:p + 1, :]
        proj_ref[0, p * NOUT:(p + 1) * NOUT, :] = jnp.concatenate(
            [prx * np_ + mxp, pry * np_ + myp, prz * np_ + mzp], axis=1)
        simp_ref[0, p * NOUT:(p + 1) * NOUT, :] = jnp.concatenate(
            [qxp * np_ + mxp, qyp * np_ + myp, qzp * np_ + mzp], axis=1)


@jax.jit
def kernel(xyz, w1_0, b1_0, w1_1, b1_1, w1_2, b1_2, w1_3, b1_3, w1_4, b1_4,
           w2_0, b2_0, w2_1, b2_1, w2_2, b2_2, w2_3, b2_3, sigma):
    col = lambda b: b.reshape(-1, 1)
    w23 = w2_3.reshape(256, NOUT, 3)
    b23 = b2_3.reshape(NOUT, 3)
    tt = (sigma ** 2 + 1e-4).reshape(1, 1)
    full2 = lambda a: pl.BlockSpec(a.shape, lambda b: (0,) * a.ndim)
    operands = [
        w1_0.T, col(b1_0), w1_1.T, col(b1_1), w1_2.T, col(b1_2),
        w1_3.T, col(b1_3), w1_4.T, col(b1_4),
        w2_0.T, col(b2_0), w2_1.T, col(b2_1), w2_2.T, col(b2_2),
        w23[:, :, 0].T, w23[:, :, 1].T, w23[:, :, 2].T,
        b23[:, 0:1], b23[:, 1:2], b23[:, 2:3], tt,
    ]
    proj, simp, seeds = pl.pallas_call(
        _samplenet_kernel,
        grid=(B,),
        in_specs=[pl.BlockSpec((1, 3, N), lambda b: (b, 0, 0))]
        + [full2(a) for a in operands],
        out_specs=[
            pl.BlockSpec((1, NPATCH * NOUT, 3), lambda b: (b, 0, 0)),
            pl.BlockSpec((1, NPATCH * NOUT, 3), lambda b: (b, 0, 0)),
            pl.BlockSpec((1, NPATCH, 3), lambda b: (b, 0, 0)),
        ],
        out_shape=[
            jax.ShapeDtypeStruct((B, NPATCH * NOUT, 3), F32),
            jax.ShapeDtypeStruct((B, NPATCH * NOUT, 3), F32),
            jax.ShapeDtypeStruct((B, NPATCH, 3), F32),
        ],
    )(xyz, *operands)
    return proj, simp, seeds


# speedup vs baseline: 2.4174x; 2.4174x over previous
import functools

import jax
import jax.numpy as jnp
from jax.experimental import pallas as pl

B = 64
N = 2048
NPATCH = 4
NSAMPLE = 256
NOUT = 16
GROUP = 7
BISECT_ITERS = 50
F32 = jnp.float32


def _bisect_kth(dvals, k, iters=BISECT_ITERS):
    lo = jnp.zeros((dvals.shape[0], 1), dtype=F32)
    hi = jnp.max(dvals, axis=1, keepdims=True) + 1.0

    def body(_, carry):
        lo, hi = carry
        mid = 0.5 * (lo + hi)
        cnt = jnp.sum((dvals <= mid).astype(F32), axis=1, keepdims=True)
        pred = cnt >= k
        return (jnp.where(pred, lo, mid), jnp.where(pred, mid, hi))

    lo, hi = jax.lax.fori_loop(0, iters, body, (lo, hi))
    return hi


def _samplenet_kernel(xyz_ref,
                      w0t, b0c, w1t, b1c, w2t, b2c, w3t, b3c, w4t, b4c,
                      v0t, c0c, v1t, c1c, v2t, c2c, wxt, wyt, wzt,
                      bxc, byc, bzc, tt_ref,
                      proj_ref, simp_ref, seeds_ref):
    (w0t, b0c, w1t, b1c, w2t, b2c, w3t, b3c, w4t, b4c,
     v0t, c0c, v1t, c1c, v2t, c2c, wxt, wyt, wzt, bxc, byc, bzc) = (
        r[...] for r in (w0t, b0c, w1t, b1c, w2t, b2c, w3t, b3c, w4t, b4c,
                         v0t, c0c, v1t, c1c, v2t, c2c, wxt, wyt, wzt,
                         bxc, byc, bzc))
    x = xyz_ref[0, 0:1, :]
    y = xyz_ref[0, 1:2, :]
    z = xyz_ref[0, 2:3, :]
    iota = jax.lax.broadcasted_iota(jnp.int32, (1, N), 1)

    oh = (iota == 0).astype(F32)
    sel = [oh]
    lx = jnp.sum(x * oh, axis=1, keepdims=True)
    ly = jnp.sum(y * oh, axis=1, keepdims=True)
    lz = jnp.sum(z * oh, axis=1, keepdims=True)
    dists = jnp.full((1, N), 1e10, dtype=F32)
    for _ in range(NPATCH - 1):
        d = (x - lx) ** 2 + (y - ly) ** 2 + (z - lz) ** 2
        dists = jnp.minimum(dists, d)
        m = jnp.max(dists, axis=1, keepdims=True)
        idx = jnp.min(jnp.where(dists >= m, iota, N), axis=1, keepdims=True)
        oh = (iota == idx).astype(F32)
        sel.append(oh)
        lx = jnp.sum(x * oh, axis=1, keepdims=True)
        ly = jnp.sum(y * oh, axis=1, keepdims=True)
        lz = jnp.sum(z * oh, axis=1, keepdims=True)
    selm = jnp.concatenate(sel, axis=0)
    sx = jnp.sum(selm * x, axis=1, keepdims=True)
    sy = jnp.sum(selm * y, axis=1, keepdims=True)
    sz = jnp.sum(selm * z, axis=1, keepdims=True)
    seeds_ref[0] = jnp.concatenate([sx, sy, sz], axis=1)

    d2 = (sx - x) ** 2 + (sy - y) ** 2 + (sz - z) ** 2
    mask = (d2 <= _bisect_kth(d2, NSAMPLE)).astype(F32)

    inv = 1.0 / NSAMPLE
    mx = jnp.sum(mask * x, axis=1, keepdims=True) * inv
    my = jnp.sum(mask * y, axis=1, keepdims=True) * inv
    mz = jnp.sum(mask * z, axis=1, keepdims=True) * inv
    ex = x - mx
    ey = y - my
    ez = z - mz
    r2 = ex * ex + ey * ey + ez * ez
    norm = jnp.sqrt(jnp.max(jnp.where(mask > 0, r2, 0.0), axis=1,
                            keepdims=True) + 1e-12)
    xn = ex / norm
    yn = ey / norm
    zn = ez / norm

    tt = tt_ref[0, 0]

    pooled = []
    for p in range(NPATCH):
        xr = xn[p:p + 1, :]
        yr = yn[p:p + 1, :]
        zr = zn[p:p + 1, :]
        h = (w0t[:, 0:1] * xr + w0t[:, 1:2] * yr + w0t[:, 2:3] * zr + b0c)
        h = jnp.maximum(h, 0.0)
        h = jnp.maximum(jnp.dot(w1t, h, preferred_element_type=F32) + b1c, 0.0)
        h = jnp.maximum(jnp.dot(w2t, h, preferred_element_type=F32) + b2c, 0.0)
        h = jnp.maximum(jnp.dot(w3t, h, preferred_element_type=F32) + b3c, 0.0)
        h = jnp.maximum(jnp.dot(w4t, h, preferred_element_type=F32) + b4c, 0.0)
        pooled.append(jnp.max(jnp.where(mask[p:p + 1, :] > 0, h, 0.0),
                              axis=1, keepdims=True))
    f = jnp.concatenate(pooled, axis=1)

    g = jnp.maximum(jnp.dot(v0t, f, preferred_element_type=F32) + c0c, 0.0)
    g = jnp.maximum(jnp.dot(v1t, g, preferred_element_type=F32) + c1c, 0.0)
    g = jnp.maximum(jnp.dot(v2t, g, preferred_element_type=F32) + c2c, 0.0)
    qx = jnp.dot(wxt, g, preferred_element_type=F32) + bxc
    qy = jnp.dot(wyt, g, preferred_element_type=F32) + byc
    qz = jnp.dot(wzt, g, preferred_element_type=F32) + bzc

    def rep16(a):
        return jnp.concatenate(
            [jnp.broadcast_to(a[p:p + 1, :], (NOUT, a.shape[1]))
             for p in range(NPATCH)], axis=0)

    def colcat(a):
        return jnp.concatenate([a[:, p:p + 1] for p in range(NPATCH)], axis=0)

    XN = rep16(xn)
    YN = rep16(yn)
    ZN = rep16(zn)
    MP = rep16(mask)
    QX = colcat(qx)
    QY = colcat(qy)
    QZ = colcat(qz)
    d2p = (QX - XN) ** 2 + (QY - YN) ** 2 + (QZ - ZN) ** 2
    dm = jnp.where(MP > 0, d2p, 1e30)
    iota64 = jax.lax.broadcasted_iota(jnp.int32, (NPATCH * NOUT, N), 1)
    work = dm
    gm = jnp.zeros((NPATCH * NOUT, N), dtype=jnp.bool_)
    dmin = None
    for i in range(GROUP):
        m_i = jnp.min(work, axis=1, keepdims=True)
        if i == 0:
            dmin = m_i
        idxm = jnp.min(jnp.where(work <= m_i, iota64, N), axis=1, keepdims=True)
        hit = iota64 == idxm
        gm = gm | hit
        if i < GROUP - 1:
            work = jnp.where(hit, 1e30, work)
    e = jnp.where(gm, jnp.exp((dmin - d2p) / tt), 0.0)
    s = jnp.sum(e, axis=1, keepdims=True)
    wgt = e / s
    prx = jnp.sum(wgt * XN, axis=1, keepdims=True)
    pry = jnp.sum(wgt * YN, axis=1, keepdims=True)
    prz = jnp.sum(wgt * ZN, axis=1, keepdims=True)
    NORM = rep16(norm)
    MX = rep16(mx)
    MY = rep16(my)
    MZ = rep16(mz)
    proj_ref[0] = jnp.concatenate(
        [prx * NORM + MX, pry * NORM + MY, prz * NORM + MZ], axis=1)
    simp_ref[0] = jnp.concatenate(
        [QX * NORM + MX, QY * NORM + MY, QZ * NORM + MZ], axis=1)


@jax.jit
def kernel(xyz, w1_0, b1_0, w1_1, b1_1, w1_2, b1_2, w1_3, b1_3, w1_4, b1_4,
           w2_0, b2_0, w2_1, b2_1, w2_2, b2_2, w2_3, b2_3, sigma):
    col = lambda b: b.reshape(-1, 1)
    w23 = w2_3.reshape(256, NOUT, 3)
    b23 = b2_3.reshape(NOUT, 3)
    tt = (sigma ** 2 + 1e-4).reshape(1, 1)
    full2 = lambda a: pl.BlockSpec(a.shape, lambda b: (0,) * a.ndim)
    operands = [
        w1_0.T, col(b1_0), w1_1.T, col(b1_1), w1_2.T, col(b1_2),
        w1_3.T, col(b1_3), w1_4.T, col(b1_4),
        w2_0.T, col(b2_0), w2_1.T, col(b2_1), w2_2.T, col(b2_2),
        w23[:, :, 0].T, w23[:, :, 1].T, w23[:, :, 2].T,
        b23[:, 0:1], b23[:, 1:2], b23[:, 2:3], tt,
    ]
    proj, simp, seeds = pl.pallas_call(
        _samplenet_kernel,
        grid=(B,),
        in_specs=[pl.BlockSpec((1, 3, N), lambda b: (b, 0, 0))]
        + [full2(a) for a in operands],
        out_specs=[
            pl.BlockSpec((1, NPATCH * NOUT, 3), lambda b: (b, 0, 0)),
            pl.BlockSpec((1, NPATCH * NOUT, 3), lambda b: (b, 0, 0)),
            pl.BlockSpec((1, NPATCH, 3), lambda b: (b, 0, 0)),
        ],
        out_shape=[
            jax.ShapeDtypeStruct((B, NPATCH * NOUT, 3), F32),
            jax.ShapeDtypeStruct((B, NPATCH * NOUT, 3), F32),
            jax.ShapeDtypeStruct((B, NPATCH, 3), F32),
        ],
    )(xyz, *operands)
    return proj, simp, seeds


# speedup vs baseline: 2.8518x; 1.1797x over previous
import functools

import jax
import jax.numpy as jnp
from jax.experimental import pallas as pl

B = 64
N = 2048
NPATCH = 4
NSAMPLE = 256
NOUT = 16
GROUP = 7
BISECT_ITERS = 50
F32 = jnp.float32


def _kth_smallest(dvals, k):
    rows = dvals.shape[0]
    jcol = (jax.lax.broadcasted_iota(jnp.int32, (15, 1), 0) + 1).astype(F32)
    los, his = [], []
    for p in range(rows):
        dp = jnp.broadcast_to(dvals[p:p + 1, :], (15, dvals.shape[1]))
        lo = jnp.zeros((1, 1), dtype=F32)
        hi = jnp.max(dvals[p:p + 1, :], axis=1, keepdims=True) + 1.0

        def mbody(_, carry, dp=dp):
            lo, hi = carry
            t = lo + (hi - lo) * 0.0625 * jcol
            cnt = jnp.sum((dp <= t).astype(F32), axis=1, keepdims=True)
            lo2 = jnp.max(jnp.where(cnt < k, t, lo), axis=0, keepdims=True)
            hi2 = jnp.min(jnp.where(cnt >= k, t, hi), axis=0, keepdims=True)
            return (lo2, hi2)

        lo, hi = jax.lax.fori_loop(0, 10, mbody, (lo, hi))
        los.append(lo)
        his.append(hi)
    lo = jnp.concatenate(los, axis=0)
    hi = jnp.concatenate(his, axis=0)

    def body(_, carry):
        lo, hi = carry
        mid = 0.5 * (lo + hi)
        cnt = jnp.sum((dvals <= mid).astype(F32), axis=1, keepdims=True)
        pred = cnt >= k
        return (jnp.where(pred, lo, mid), jnp.where(pred, mid, hi))

    lo, hi = jax.lax.fori_loop(0, 8, body, (lo, hi))
    return hi


def _samplenet_kernel(xyz_ref,
                      w0t, b0c, w1t, b1c, w2t, b2c, w3t, b3c, w4t, b4c,
                      v0t, c0c, v1t, c1c, v2t, c2c, wxt, wyt, wzt,
                      bxc, byc, bzc, tt_ref,
                      proj_ref, simp_ref, seeds_ref):
    (w0t, b0c, w1t, b1c, w2t, b2c, w3t, b3c, w4t, b4c,
     v0t, c0c, v1t, c1c, v2t, c2c, wxt, wyt, wzt, bxc, byc, bzc) = (
        r[...] for r in (w0t, b0c, w1t, b1c, w2t, b2c, w3t, b3c, w4t, b4c,
                         v0t, c0c, v1t, c1c, v2t, c2c, wxt, wyt, wzt,
                         bxc, byc, bzc))
    x = xyz_ref[0, 0:1, :]
    y = xyz_ref[0, 1:2, :]
    z = xyz_ref[0, 2:3, :]
    iota = jax.lax.broadcasted_iota(jnp.int32, (1, N), 1)

    oh = (iota == 0).astype(F32)
    sel = [oh]
    lx = jnp.sum(x * oh, axis=1, keepdims=True)
    ly = jnp.sum(y * oh, axis=1, keepdims=True)
    lz = jnp.sum(z * oh, axis=1, keepdims=True)
    dists = jnp.full((1, N), 1e10, dtype=F32)
    for _ in range(NPATCH - 1):
        d = (x - lx) ** 2 + (y - ly) ** 2 + (z - lz) ** 2
        dists = jnp.minimum(dists, d)
        m = jnp.max(dists, axis=1, keepdims=True)
        idx = jnp.min(jnp.where(dists >= m, iota, N), axis=1, keepdims=True)
        oh = (iota == idx).astype(F32)
        sel.append(oh)
        lx = jnp.sum(x * oh, axis=1, keepdims=True)
        ly = jnp.sum(y * oh, axis=1, keepdims=True)
        lz = jnp.sum(z * oh, axis=1, keepdims=True)
    selm = jnp.concatenate(sel, axis=0)
    sx = jnp.sum(selm * x, axis=1, keepdims=True)
    sy = jnp.sum(selm * y, axis=1, keepdims=True)
    sz = jnp.sum(selm * z, axis=1, keepdims=True)
    seeds_ref[0] = jnp.concatenate([sx, sy, sz], axis=1)

    d2 = (sx - x) ** 2 + (sy - y) ** 2 + (sz - z) ** 2
    mask = (d2 <= _kth_smallest(d2, NSAMPLE)).astype(F32)

    inv = 1.0 / NSAMPLE
    mx = jnp.sum(mask * x, axis=1, keepdims=True) * inv
    my = jnp.sum(mask * y, axis=1, keepdims=True) * inv
    mz = jnp.sum(mask * z, axis=1, keepdims=True) * inv
    ex = x - mx
    ey = y - my
    ez = z - mz
    r2 = ex * ex + ey * ey + ez * ez
    norm = jnp.sqrt(jnp.max(jnp.where(mask > 0, r2, 0.0), axis=1,
                            keepdims=True) + 1e-12)
    xn = ex / norm
    yn = ey / norm
    zn = ez / norm

    tt = tt_ref[0, 0]

    pooled = []
    for p in range(NPATCH):
        xr = xn[p:p + 1, :]
        yr = yn[p:p + 1, :]
        zr = zn[p:p + 1, :]
        h = (w0t[:, 0:1] * xr + w0t[:, 1:2] * yr + w0t[:, 2:3] * zr + b0c)
        h = jnp.maximum(h, 0.0)
        h = jnp.maximum(jnp.dot(w1t, h, preferred_element_type=F32) + b1c, 0.0)
        h = jnp.maximum(jnp.dot(w2t, h, preferred_element_type=F32) + b2c, 0.0)
        h = jnp.maximum(jnp.dot(w3t, h, preferred_element_type=F32) + b3c, 0.0)
        h = jnp.maximum(jnp.dot(w4t, h, preferred_element_type=F32) + b4c, 0.0)
        pooled.append(jnp.max(jnp.where(mask[p:p + 1, :] > 0, h, 0.0),
                              axis=1, keepdims=True))
    f = jnp.concatenate(pooled, axis=1)

    g = jnp.maximum(jnp.dot(v0t, f, preferred_element_type=F32) + c0c, 0.0)
    g = jnp.maximum(jnp.dot(v1t, g, preferred_element_type=F32) + c1c, 0.0)
    g = jnp.maximum(jnp.dot(v2t, g, preferred_element_type=F32) + c2c, 0.0)
    qx = jnp.dot(wxt, g, preferred_element_type=F32) + bxc
    qy = jnp.dot(wyt, g, preferred_element_type=F32) + byc
    qz = jnp.dot(wzt, g, preferred_element_type=F32) + bzc

    def rep16(a):
        return jnp.concatenate(
            [jnp.broadcast_to(a[p:p + 1, :], (NOUT, a.shape[1]))
             for p in range(NPATCH)], axis=0)

    def colcat(a):
        return jnp.concatenate([a[:, p:p + 1] for p in range(NPATCH)], axis=0)

    XN = rep16(xn)
    YN = rep16(yn)
    ZN = rep16(zn)
    MP = rep16(mask)
    QX = colcat(qx)
    QY = colcat(qy)
    QZ = colcat(qz)
    d2p = (QX - XN) ** 2 + (QY - YN) ** 2 + (QZ - ZN) ** 2
    dm = jnp.where(MP > 0, d2p, 1e30)
    iota64 = jax.lax.broadcasted_iota(jnp.int32, (NPATCH * NOUT, N), 1)
    work = dm
    gm = jnp.zeros((NPATCH * NOUT, N), dtype=jnp.bool_)
    dmin = None
    for i in range(GROUP):
        m_i = jnp.min(work, axis=1, keepdims=True)
        if i == 0:
            dmin = m_i
        idxm = jnp.min(jnp.where(work <= m_i, iota64, N), axis=1, keepdims=True)
        hit = iota64 == idxm
        gm = gm | hit
        if i < GROUP - 1:
            work = jnp.where(hit, 1e30, work)
    e = jnp.where(gm, jnp.exp((dmin - d2p) / tt), 0.0)
    s = jnp.sum(e, axis=1, keepdims=True)
    wgt = e / s
    prx = jnp.sum(wgt * XN, axis=1, keepdims=True)
    pry = jnp.sum(wgt * YN, axis=1, keepdims=True)
    prz = jnp.sum(wgt * ZN, axis=1, keepdims=True)
    NORM = rep16(norm)
    MX = rep16(mx)
    MY = rep16(my)
    MZ = rep16(mz)
    proj_ref[0] = jnp.concatenate(
        [prx * NORM + MX, pry * NORM + MY, prz * NORM + MZ], axis=1)
    simp_ref[0] = jnp.concatenate(
        [QX * NORM + MX, QY * NORM + MY, QZ * NORM + MZ], axis=1)


@jax.jit
def kernel(xyz, w1_0, b1_0, w1_1, b1_1, w1_2, b1_2, w1_3, b1_3, w1_4, b1_4,
           w2_0, b2_0, w2_1, b2_1, w2_2, b2_2, w2_3, b2_3, sigma):
    col = lambda b: b.reshape(-1, 1)
    w23 = w2_3.reshape(256, NOUT, 3)
    b23 = b2_3.reshape(NOUT, 3)
    tt = (sigma ** 2 + 1e-4).reshape(1, 1)
    full2 = lambda a: pl.BlockSpec(a.shape, lambda b: (0,) * a.ndim)
    operands = [
        w1_0.T, col(b1_0), w1_1.T, col(b1_1), w1_2.T, col(b1_2),
        w1_3.T, col(b1_3), w1_4.T, col(b1_4),
        w2_0.T, col(b2_0), w2_1.T, col(b2_1), w2_2.T, col(b2_2),
        w23[:, :, 0].T, w23[:, :, 1].T, w23[:, :, 2].T,
        b23[:, 0:1], b23[:, 1:2], b23[:, 2:3], tt,
    ]
    proj, simp, seeds = pl.pallas_call(
        _samplenet_kernel,
        grid=(B,),
        in_specs=[pl.BlockSpec((1, 3, N), lambda b: (b, 0, 0))]
        + [full2(a) for a in operands],
        out_specs=[
            pl.BlockSpec((1, NPATCH * NOUT, 3), lambda b: (b, 0, 0)),
            pl.BlockSpec((1, NPATCH * NOUT, 3), lambda b: (b, 0, 0)),
            pl.BlockSpec((1, NPATCH, 3), lambda b: (b, 0, 0)),
        ],
        out_shape=[
            jax.ShapeDtypeStruct((B, NPATCH * NOUT, 3), F32),
            jax.ShapeDtypeStruct((B, NPATCH * NOUT, 3), F32),
            jax.ShapeDtypeStruct((B, NPATCH, 3), F32),
        ],
    )(xyz, *operands)
    return proj, simp, seeds
